# Initial kernel scaffold; baseline (speedup 1.0000x reference)
import jax
import jax.numpy as jnp
from jax.experimental import pallas as pl


def kernel(x, attn_mask, emb, W_in, W_out, gru_w_ih, gru_w_hh, gru_b_ih, gru_b_hh, read_W, read_b):
    raise NotImplementedError("write your pallas kernel here")


# speedup vs baseline: 2.2301x; 2.2301x over previous
import functools

import jax
import jax.numpy as jnp
from jax import lax
from jax.experimental import pallas as pl
from jax.experimental.pallas import tpu as pltpu
from jax.experimental.pallas import tpu_sc as plsc

_V = 100000
_D = 128
_L = 200


_NC = 2
_NS = 16
_NW = _NC * _NS
_CHUNK = 128


@functools.lru_cache(maxsize=None)
def _sc_gather_kernel(n_rows, d):
    rows_per_w = n_rows // _NW
    n_chunk = rows_per_w // _CHUNK
    mesh = plsc.VectorSubcoreMesh(
        core_axis_name="c", subcore_axis_name="s",
        num_cores=_NC, num_subcores=_NS)

    @functools.partial(
        pl.kernel, mesh=mesh,
        out_type=jax.ShapeDtypeStruct((n_rows, d), jnp.float32),
        scratch_types=[
            pltpu.VMEM((_CHUNK,), jnp.int32),
            pltpu.VMEM((_CHUNK, d), jnp.float32),
            pltpu.SemaphoreType.DMA,
        ],
    )
    def gather(emb_hbm, idx_hbm, out_hbm, idx_v, rows_v, sem):
        wid = lax.axis_index("s") * _NC + lax.axis_index("c")
        base = wid * rows_per_w

        def body(c, carry):
            off = base + c * _CHUNK
            pltpu.sync_copy(idx_hbm.at[pl.ds(off, _CHUNK)], idx_v)
            pltpu.async_copy(emb_hbm.at[idx_v], rows_v, sem).wait()
            pltpu.sync_copy(rows_v, out_hbm.at[pl.ds(off, _CHUNK)])
            return carry

        lax.fori_loop(0, n_chunk, body, 0)

    return gather


def _gather_rows(emb, idx):
    return _sc_gather_kernel(idx.shape[0], emb.shape[1])(emb, idx)


def _graph_body(xr_ref, xc_ref, e_ref, win_ref, wout_ref, wih_ref, whh_ref,
                bih_ref, bhh_ref, rw_ref, rb_ref, hread_ref, flag_ref):
    f32 = jnp.float32
    L, D = _L, _D

    sr = xr_ref[0]
    sc = xc_ref[0]
    mrf = (sr != 0).astype(f32)
    mcf = (sc != 0).astype(f32)

    ic = lax.broadcasted_iota(jnp.int32, (L, L), 0)
    ir = lax.broadcasted_iota(jnp.int32, (L, L), 1)

    le = (ic <= ir).astype(f32)
    rank_r = jnp.dot(mrf, le, preferred_element_type=f32)
    ge = (ic >= ir).astype(f32)
    rank_c = jnp.dot(ge, mcf, preferred_element_type=f32)
    n = jnp.max(rank_r)

    strict = (ic > ir).astype(f32)

    P = (mcf * mrf) * strict * (jnp.abs((rank_c - 1.0) - rank_r) < 0.5).astype(f32)

    E = e_ref[...]
    dotT = lambda a, b: lax.dot_general(
        a, b, (((1,), (1,)), ((), ())), preferred_element_type=f32)

    Hin = dotT(E, win_ref[...])
    Hout = dotT(E, wout_ref[...])
    Hpred = jnp.dot(P, Hout, preferred_element_type=f32)

    Qf = (sc == sr).astype(f32)
    em_r = mrf * (rank_r >= 1.5).astype(f32)
    Qe = Qf * em_r
    agg = jnp.dot(Qe, Hpred, preferred_element_type=f32)
    deg = jnp.maximum(jnp.sum(Qe, axis=1, keepdims=True), 1.0)
    xg = Hin + agg / deg

    gi = dotT(xg, wih_ref[...]) + bih_ref[...]
    gh = dotT(E, whh_ref[...]) + bhh_ref[...]
    r = jax.nn.sigmoid(gi[:, :D] + gh[:, :D])
    z = jax.nn.sigmoid(gi[:, D:2 * D] + gh[:, D:2 * D])
    ng = jnp.tanh(gi[:, 2 * D:] + r * gh[:, 2 * D:])
    H = (1.0 - z) * ng + z * E

    ol = mrf * (jnp.abs(rank_r - n) < 0.5).astype(f32)
    h_last = jnp.dot(ol, H, preferred_element_type=f32)

    prev_same = jnp.sum(Qf * mrf * strict, axis=1, keepdims=True)
    w_c = mcf * (prev_same < 0.5).astype(f32)
    logit = jnp.sum(H * h_last, axis=1, keepdims=True)
    lm = jnp.where(w_c > 0.5, logit, -1e30)
    ex = jnp.exp(lm - jnp.max(lm)) * w_c
    att = ex / jnp.maximum(jnp.sum(ex), 1e-30)
    local = jnp.sum(att * H, axis=0, keepdims=True)

    pre = (dotT(h_last, rw_ref[:, :D]) + dotT(local, rw_ref[:, D:2 * D])
           + rb_ref[...])
    hread_ref[0] = jnp.tanh(pre)
    flag_ref[0] = jnp.full((1, 1), 1.0, f32) * (n >= 1.5).astype(f32)


_BM = 256
_BN = 4096


def _logits_body(h_ref, emb_ref, flag_ref, out_ref):
    j = pl.program_id(1)
    acc = lax.dot_general(h_ref[...], emb_ref[...],
                          (((1,), (1,)), ((), ())),
                          preferred_element_type=jnp.float32)
    col = j * out_ref.shape[1] + lax.broadcasted_iota(
        jnp.int32, acc.shape, 1)
    default = jnp.where(col == 0, 0.0, -1e9)
    out_ref[...] = jnp.where(flag_ref[...] > 0.5, acc, default)


def kernel(x, attn_mask, emb, W_in, W_out, gru_w_ih, gru_w_hh,
           gru_b_ih, gru_b_hh, read_W, read_b):
    B, L = x.shape
    V, D = emb.shape
    f32 = jnp.float32

    E0 = _gather_rows(emb, x.reshape(-1))

    bih = gru_b_ih.reshape(1, 3 * D).astype(f32)
    bhh = gru_b_hh.reshape(1, 3 * D).astype(f32)
    rb = read_b.reshape(1, D).astype(f32)

    hread, flag = pl.pallas_call(
        _graph_body,
        grid=(B,),
        in_specs=[
            pl.BlockSpec((1, 1, L), lambda i: (i, 0, 0)),
            pl.BlockSpec((1, L, 1), lambda i: (i, 0, 0)),
            pl.BlockSpec((L, D), lambda i: (i, 0)),
            pl.BlockSpec((D, D), lambda i: (0, 0)),
            pl.BlockSpec((D, D), lambda i: (0, 0)),
            pl.BlockSpec((3 * D, D), lambda i: (0, 0)),
            pl.BlockSpec((3 * D, D), lambda i: (0, 0)),
            pl.BlockSpec((1, 3 * D), lambda i: (0, 0)),
            pl.BlockSpec((1, 3 * D), lambda i: (0, 0)),
            pl.BlockSpec((D, 3 * D), lambda i: (0, 0)),
            pl.BlockSpec((1, D), lambda i: (0, 0)),
        ],
        out_specs=[
            pl.BlockSpec((1, 1, D), lambda i: (i, 0, 0)),
            pl.BlockSpec((1, 1, 1), lambda i: (i, 0, 0)),
        ],
        out_shape=[
            jax.ShapeDtypeStruct((B, 1, D), f32),
            jax.ShapeDtypeStruct((B, 1, 1), f32),
        ],
    )(x.reshape(B, 1, L), x.reshape(B, L, 1), E0,
      W_in, W_out, gru_w_ih, gru_w_hh, bih, bhh, read_W, rb)
    hread = hread.reshape(B, D)
    flag = flag.reshape(B, 1)

    bm = min(_BM, B)
    bn = min(_BN, V)
    logits = pl.pallas_call(
        _logits_body,
        grid=(B // bm, pl.cdiv(V, bn)),
        in_specs=[
            pl.BlockSpec((bm, D), lambda i, j: (i, 0)),
            pl.BlockSpec((bn, D), lambda i, j: (j, 0)),
            pl.BlockSpec((bm, 1), lambda i, j: (i, 0)),
        ],
        out_specs=pl.BlockSpec((bm, bn), lambda i, j: (i, j)),
        out_shape=jax.ShapeDtypeStruct((B, V), f32),
    )(hread, emb, flag)

    return logits


# speedup vs baseline: 3.8408x; 1.7223x over previous
import functools

import jax
import jax.numpy as jnp
from jax import lax
from jax.experimental import pallas as pl
from jax.experimental.pallas import tpu as pltpu
from jax.experimental.pallas import tpu_sc as plsc

_V = 100000
_D = 128
_L = 200


_NC = 2
_NS = 16
_NW = _NC * _NS
_CHUNK = 128


@functools.lru_cache(maxsize=None)
def _sc_gather_kernel(n_rows, d):
    rows_per_w = n_rows // _NW
    n_chunk = rows_per_w // _CHUNK
    mesh = plsc.VectorSubcoreMesh(
        core_axis_name="c", subcore_axis_name="s",
        num_cores=_NC, num_subcores=_NS)

    @functools.partial(
        pl.kernel, mesh=mesh,
        out_type=jax.ShapeDtypeStruct((n_rows, d), jnp.float32),
        scratch_types=[
            pltpu.VMEM((_CHUNK,), jnp.int32),
            pltpu.VMEM((_CHUNK, d), jnp.float32),
            pltpu.SemaphoreType.DMA,
        ],
    )
    def gather(emb_hbm, idx_hbm, out_hbm, idx_v, rows_v, sem):
        wid = lax.axis_index("s") * _NC + lax.axis_index("c")
        base = wid * rows_per_w

        def body(c, carry):
            off = base + c * _CHUNK
            pltpu.sync_copy(idx_hbm.at[pl.ds(off, _CHUNK)], idx_v)
            pltpu.async_copy(emb_hbm.at[idx_v], rows_v, sem).wait()
            pltpu.sync_copy(rows_v, out_hbm.at[pl.ds(off, _CHUNK)])
            return carry

        lax.fori_loop(0, n_chunk, body, 0)

    return gather


def _gather_rows(emb, idx):
    return _sc_gather_kernel(idx.shape[0], emb.shape[1])(emb, idx)


_SB = 8


def _dotT(a, b):
    return lax.dot_general(a, b, (((1,), (1,)), ((), ())),
                           preferred_element_type=jnp.float32)


def _graph_body(x_ref, xc_ref, e_ref, wcat_ref, wih_ref,
                bih_ref, bhh_ref, rw_ref, rb_ref, hread_ref, flag_ref):
    f32 = jnp.float32
    L, D, SB = _L, _D, _SB

    ic = lax.broadcasted_iota(jnp.int32, (L, L), 0)
    ir = lax.broadcasted_iota(jnp.int32, (L, L), 1)
    le = (ic <= ir).astype(f32)
    ge = (ic >= ir).astype(f32)
    strict = (ic > ir).astype(f32)

    xb = x_ref[...]
    mb = (xb != 0).astype(f32)
    rank_b = jnp.dot(mb, le, preferred_element_type=f32)
    nvec = jnp.max(rank_b, axis=1, keepdims=True)

    Eb = e_ref[...]
    Hh = _dotT(Eb, wcat_ref[...])

    xgs, wcs = [], []
    for s in range(SB):
        sr = xb[s:s + 1, :]
        sc = xc_ref[s * L:(s + 1) * L, :]
        mrf = mb[s:s + 1, :]
        mcf = (sc != 0).astype(f32)
        rank_r = rank_b[s:s + 1, :]
        rank_c = jnp.dot(ge, mcf, preferred_element_type=f32)

        P = (mcf * mrf) * strict * (
            jnp.abs((rank_c - 1.0) - rank_r) < 0.5).astype(f32)
        Qf = (sc == sr).astype(f32)
        em_r = mrf * (rank_r >= 1.5).astype(f32)
        Qe = Qf * em_r

        Hout_s = Hh[s * L:(s + 1) * L, D:2 * D]
        Hpred = jnp.dot(P, Hout_s, preferred_element_type=f32)
        agg = jnp.dot(Qe, Hpred, preferred_element_type=f32)
        deg = jnp.maximum(jnp.sum(Qe, axis=1, keepdims=True), 1.0)
        xgs.append(Hh[s * L:(s + 1) * L, :D] + agg / deg)

        prev_same = jnp.sum(Qf * mrf * strict, axis=1, keepdims=True)
        wcs.append(mcf * (prev_same < 0.5).astype(f32))

    xgb = jnp.concatenate(xgs, axis=0)
    gi = _dotT(xgb, wih_ref[...]) + bih_ref[...]
    gh = Hh[:, 2 * D:] + bhh_ref[...]
    r = jax.nn.sigmoid(gi[:, :D] + gh[:, :D])
    z = jax.nn.sigmoid(gi[:, D:2 * D] + gh[:, D:2 * D])
    ng = jnp.tanh(gi[:, 2 * D:] + r * gh[:, 2 * D:])
    Hb = (1.0 - z) * ng + z * Eb

    hls, los = [], []
    for s in range(SB):
        H = Hb[s * L:(s + 1) * L, :]
        mrf = mb[s:s + 1, :]
        rank_r = rank_b[s:s + 1, :]
        ol = mrf * (jnp.abs(rank_r - nvec[s:s + 1, :]) < 0.5).astype(f32)
        h_last = jnp.dot(ol, H, preferred_element_type=f32)
        w_c = wcs[s]
        logit = jnp.sum(H * h_last, axis=1, keepdims=True)
        lm = jnp.where(w_c > 0.5, logit, -1e30)
        ex = jnp.exp(lm - jnp.max(lm)) * w_c
        att = ex / jnp.maximum(jnp.sum(ex), 1e-30)
        hls.append(h_last)
        los.append(jnp.sum(att * H, axis=0, keepdims=True))

    hl8 = jnp.concatenate(hls, axis=0)
    lo8 = jnp.concatenate(los, axis=0)
    pre = (_dotT(hl8, rw_ref[:, :D]) + _dotT(lo8, rw_ref[:, D:2 * D])
           + rb_ref[...])
    hread_ref[...] = jnp.tanh(pre)
    flag_ref[...] = (nvec >= 1.5).astype(f32)


_BN = 2048


def _logits_body(h_ref, emb_ref, flag_ref, out_ref):
    j = pl.program_id(0)
    acc = _dotT(h_ref[...], emb_ref[...])
    col = j * out_ref.shape[1] + lax.broadcasted_iota(
        jnp.int32, acc.shape, 1)
    default = jnp.where(col == 0, 0.0, -1e9)
    out_ref[...] = jnp.where(flag_ref[...] > 0.5, acc, default)


def kernel(x, attn_mask, emb, W_in, W_out, gru_w_ih, gru_w_hh,
           gru_b_ih, gru_b_hh, read_W, read_b):
    B, L = x.shape
    V, D = emb.shape
    f32 = jnp.float32

    E0 = _gather_rows(emb, x.reshape(-1))

    bih = gru_b_ih.reshape(1, 3 * D).astype(f32)
    bhh = gru_b_hh.reshape(1, 3 * D).astype(f32)
    rb = read_b.reshape(1, D).astype(f32)
    wcat = jnp.concatenate([W_in, W_out, gru_w_hh], axis=0)

    SB = _SB
    hread, flag = pl.pallas_call(
        _graph_body,
        grid=(B // SB,),
        in_specs=[
            pl.BlockSpec((SB, L), lambda i: (i, 0)),
            pl.BlockSpec((SB * L, 1), lambda i: (i, 0)),
            pl.BlockSpec((SB * L, D), lambda i: (i, 0)),
            pl.BlockSpec((5 * D, D), lambda i: (0, 0)),
            pl.BlockSpec((3 * D, D), lambda i: (0, 0)),
            pl.BlockSpec((1, 3 * D), lambda i: (0, 0)),
            pl.BlockSpec((1, 3 * D), lambda i: (0, 0)),
            pl.BlockSpec((D, 3 * D), lambda i: (0, 0)),
            pl.BlockSpec((1, D), lambda i: (0, 0)),
        ],
        out_specs=[
            pl.BlockSpec((SB, D), lambda i: (i, 0)),
            pl.BlockSpec((SB, 1), lambda i: (i, 0)),
        ],
        out_shape=[
            jax.ShapeDtypeStruct((B, D), f32),
            jax.ShapeDtypeStruct((B, 1), f32),
        ],
    )(x, x.reshape(B * L, 1), E0,
      wcat, gru_w_ih, bih, bhh, read_W, rb)

    bn = min(_BN, V)
    logits = pl.pallas_call(
        _logits_body,
        grid=(pl.cdiv(V, bn),),
        in_specs=[
            pl.BlockSpec((B, D), lambda j: (0, 0)),
            pl.BlockSpec((bn, D), lambda j: (j, 0)),
            pl.BlockSpec((B, 1), lambda j: (0, 0)),
        ],
        out_specs=pl.BlockSpec((B, bn), lambda j: (0, j)),
        out_shape=jax.ShapeDtypeStruct((B, V), f32),
    )(hread, emb, flag)

    return logits
